# baseline (speedup 1.0000x reference)
import jax
import jax.numpy as jnp
from jax import lax
from jax.experimental import pallas as pl
from jax.experimental.pallas import tpu as pltpu
from jax.experimental.pallas import tpu_sc as plsc

B = 16384
D = 32
VU = 1_000_000
VQ = 100_000
L = 16
NC, NS = 2, 16
NW = NC * NS
RPW = B // NW
CH = 128
NCH = RPW // CH
GROUPS = RPW // L


def _sc_body(uid_hbm, qid_hbm, ut_hbm, qt_hbm, w_hbm,
             out_hbm,
             iu, iq, wxu, wxq, rows_ut, rows_qt, wv, outbuf, sem):
    c_ax = lax.axis_index("c")
    s_ax = lax.axis_index("s")
    wid = s_ax * NC + c_ax
    base = wid * RPW

    pltpu.sync_copy(uid_hbm.at[pl.ds(base, RPW)], iu)
    pltpu.sync_copy(qid_hbm.at[pl.ds(base, RPW)], iq)
    pltpu.sync_copy(w_hbm, wv)

    for j in range(NCH):
        sl = pl.ds(j * CH, CH)
        iuv = iu[sl]
        iqv = iq[sl]
        for c in range(D):
            wxu[c, sl] = iuv + (c * VU)
            wxq[c, sl] = iqv + (c * VQ)

    cps = []
    for j in range(NCH):
        sl = pl.ds(j * CH, CH)
        for c in range(D):
            cps.append(pltpu.async_copy(
                ut_hbm.at[wxu.at[c, sl]], rows_ut.at[c, sl], sem))
            cps.append(pltpu.async_copy(
                qt_hbm.at[wxq.at[c, sl]], rows_qt.at[c, sl], sem))
    for cp in cps:
        cp.wait()

    wvecs = [wv[pl.ds(i * L, L)] for i in range(2 * D // L)]
    dbv = wv[pl.ds(2 * D, L)]
    lanes = lax.iota(jnp.int32, L)
    even = (lanes & 1) == 0
    half = lanes >> 1

    def group(g, carry):
        sl = pl.ds(g * L, L)
        accs = [dbv, jnp.zeros((L,), jnp.float32),
                jnp.zeros((L,), jnp.float32), jnp.zeros((L,), jnp.float32)]
        for c in range(D):
            wu = wvecs[c // L][c % L]
            wq = wvecs[(D + c) // L][(D + c) % L]
            accs[c % 4] = accs[c % 4] + rows_ut[c, sl] * wu
            accs[(c + 1) % 4] = accs[(c + 1) % 4] + rows_qt[c, sl] * wq
        d = (accs[0] + accs[1]) + (accs[2] + accs[3])
        p0 = 1.0 / (1.0 + jnp.exp(-d))
        p1 = 1.0 - p0
        lo0 = p0.at[half].get(mode="promise_in_bounds")
        lo1 = p1.at[half].get(mode="promise_in_bounds")
        hi0 = p0.at[half + 8].get(mode="promise_in_bounds")
        hi1 = p1.at[half + 8].get(mode="promise_in_bounds")
        outbuf[pl.ds(g * 2 * L, L)] = jnp.where(even, lo0, lo1)
        outbuf[pl.ds(g * 2 * L + L, L)] = jnp.where(even, hi0, hi1)
        return carry

    lax.fori_loop(0, GROUPS, group, 0, unroll=False)

    pltpu.sync_copy(outbuf, out_hbm.at[pl.ds(base * 2, RPW * 2)])


@jax.jit
def _run(user_ids, question_ids, ut_flat, qt_flat, wpk):
    mesh = plsc.VectorSubcoreMesh(core_axis_name="c", subcore_axis_name="s")
    flat = pl.kernel(
        _sc_body,
        mesh=mesh,
        out_type=jax.ShapeDtypeStruct((B * 2,), jnp.float32),
        scratch_types=[
            pltpu.VMEM((RPW,), jnp.int32),
            pltpu.VMEM((RPW,), jnp.int32),
            pltpu.VMEM((D, RPW), jnp.int32),
            pltpu.VMEM((D, RPW), jnp.int32),
            pltpu.VMEM((D, RPW), jnp.float32),
            pltpu.VMEM((D, RPW), jnp.float32),
            pltpu.VMEM((2 * D + L,), jnp.float32),
            pltpu.VMEM((RPW * 2,), jnp.float32),
            pltpu.SemaphoreType.DMA,
        ],
    )(user_ids, question_ids, ut_flat, qt_flat, wpk)
    return flat.reshape(B, 2)


def kernel(user_ids, question_ids, user_table, question_table, W, b):
    uid = user_ids.astype(jnp.int32)
    qid = question_ids.astype(jnp.int32)
    wd = W[0] - W[1]
    db = b[0] - b[1]
    wpk = jnp.concatenate([wd, jnp.full((L,), db, jnp.float32)])
    return _run(uid, qid, user_table.T.reshape(-1),
                question_table.T.reshape(-1), wpk)
